# baseline (speedup 1.0000x reference)
import jax
import jax.numpy as jnp
from jax import lax
from jax.experimental import pallas as pl
from jax.experimental.pallas import tpu as pltpu
from jax.experimental.pallas import tpu_sc as plsc

BATCH = 4096
MAX_SEQ = 200
EMBED = 64
VOCAB = 1000000
LANES = 16

_info = plsc.get_sparse_core_info()
NUM_CORES = _info.num_cores
NUM_SUBCORES = _info.num_subcores
NUM_WORKERS = NUM_CORES * NUM_SUBCORES

VECS_PER_ROW = EMBED // LANES
BBLK = BATCH // NUM_WORKERS
CGRP = EMBED // 8

NBLK = (VOCAB + 127) // 128
FULL_BLOCKS = VOCAB // 128
BLK_PER_WORKER = (NBLK + NUM_WORKERS - 1) // NUM_WORKERS
TNBUF = 4


def _transpose_body(tbl_t_hbm, out_hbm, *refs):
    wid = lax.axis_index("s") * NUM_CORES + lax.axis_index("c")
    in_bufs = refs[0:TNBUF]
    tr_bufs = refs[TNBUF:2 * TNBUF]
    isems = refs[2 * TNBUF:3 * TNBUF]
    osems = refs[3 * TNBUF:4 * TNBUF]

    def start_in(bi, b):
        blk = wid + NUM_WORKERS * bi
        pltpu.async_copy(tbl_t_hbm.at[:, pl.ds(blk * 128, 128)],
                         in_bufs[b], isems[b])

    def wait_in(b):
        pltpu.make_async_copy(
            tbl_t_hbm.at[:, pl.ds(0, 128)], in_bufs[b], isems[b]).wait()

    def start_out(bi, b):
        blk = wid + NUM_WORKERS * bi

        @pl.when(blk < FULL_BLOCKS)
        def _():
            pltpu.async_copy(tr_bufs[b],
                             out_hbm.at[pl.ds(blk * 8192, 8192)], osems[b])

        @pl.when(blk == FULL_BLOCKS)
        def _():
            pltpu.async_copy(tr_bufs[b].at[pl.ds(0, 4096)],
                             out_hbm.at[pl.ds(FULL_BLOCKS * 8192, 4096)],
                             osems[b])

    def wait_out(bi, b):
        blk = wid + NUM_WORKERS * bi

        @pl.when(blk < FULL_BLOCKS)
        def _():
            pltpu.make_async_copy(
                tr_bufs[b], out_hbm.at[pl.ds(0, 8192)], osems[b]).wait()

        @pl.when(blk == FULL_BLOCKS)
        def _():
            pltpu.make_async_copy(
                tr_bufs[b].at[pl.ds(0, 4096)],
                out_hbm.at[pl.ds(0, 4096)], osems[b]).wait()

    def transpose(b):
        in_v = in_bufs[b]
        tr_v = tr_bufs[b]

        @plsc.parallel_loop(0, 128, 1, unroll=8)
        def _row(r):
            rvec = jnp.full((LANES,), r, jnp.int32)
            base = r * EMBED
            for j in range(VECS_PER_ROW):
                cvec = lax.iota(jnp.int32, LANES) + (j * LANES)
                v = plsc.load_gather(in_v, [cvec, rvec])
                tr_v[pl.ds(base + j * LANES, LANES)] = v

    for b in range(TNBUF - 1):
        start_in(b, b)

    def blk_body(i, carry):
        for b in range(TNBUF):
            bi = TNBUF * i + b
            blk = wid + NUM_WORKERS * bi

            @pl.when(wid + NUM_WORKERS * (bi + TNBUF - 1) <= FULL_BLOCKS)
            def _():
                start_in(bi + TNBUF - 1, (b + TNBUF - 1) % TNBUF)

            @pl.when(blk <= FULL_BLOCKS)
            def _():
                wait_in(b)

                @pl.when(bi >= TNBUF)
                def _():
                    wait_out(bi - TNBUF, b)

                transpose(b)
                start_out(bi, b)
        return carry

    n_iter = (BLK_PER_WORKER + TNBUF - 1) // TNBUF
    lax.fori_loop(0, n_iter, blk_body, 0)
    maxbi = (FULL_BLOCKS - wid) // NUM_WORKERS
    for b in range(TNBUF):
        bi_b = maxbi - ((maxbi - b) % TNBUF)

        @pl.when(bi_b >= 0)
        def _():
            wait_out(bi_b, b)


@jax.jit
def _transpose_table(tbl_t):
    mesh = plsc.VectorSubcoreMesh(core_axis_name="c", subcore_axis_name="s")
    run = pl.kernel(
        _transpose_body,
        out_type=jax.ShapeDtypeStruct((VOCAB * EMBED,), jnp.float32),
        mesh=mesh,
        scratch_types=(
            [pltpu.VMEM((EMBED, 128), jnp.float32) for _ in range(TNBUF)]
            + [pltpu.VMEM((8192,), jnp.float32) for _ in range(TNBUF)]
            + [pltpu.SemaphoreType.DMA for _ in range(2 * TNBUF)]
        ),
        compiler_params=pltpu.CompilerParams(
            use_tc_tiling_on_sc=True, needs_layout_passes=False),
    )
    return run(tbl_t)


def _embed_body(table_hbm, idx_hbm, pos_hbm, out_hbm,
                idx_v, rows0, rows1, tr0, tr1, pos_v,
                gs0, gs1, os0, os1):
    wid = lax.axis_index("s") * NUM_CORES + lax.axis_index("c")
    bufs = ((rows0, tr0, gs0, os0), (rows1, tr1, gs1, os1))

    lane = lax.iota(jnp.int32, LANES)
    gvecs = [(j * LANES + lane) // 8 for j in range(VECS_PER_ROW)]
    cvecs = [(j * LANES + lane) % 8 for j in range(VECS_PER_ROW)]

    def start_gather(t, b):
        rows_v, _, gsem, _ = bufs[b]
        pltpu.async_copy(table_hbm.at[idx_v.at[t]], rows_v, gsem)

    def wait_gather(b):
        rows_v, _, gsem, _ = bufs[b]
        pltpu.make_async_copy(table_hbm.at[idx_v.at[0]], rows_v, gsem).wait()

    def start_out(t, b):
        _, tr_v, _, osem = bufs[b]
        pltpu.async_copy(tr_v, out_hbm.at[t, :, wid], osem)

    def wait_out(b):
        _, tr_v, _, osem = bufs[b]
        pltpu.make_async_copy(tr_v, out_hbm.at[0, :, 0], osem).wait()

    def compute(t, b):
        rows_v, tr_v, _, _ = bufs[b]
        pvecs = [pos_v[t, pl.ds(j * LANES, LANES)] for j in range(VECS_PER_ROW)]

        @plsc.parallel_loop(0, BBLK, 1, unroll=8)
        def _row(r):
            rvec = jnp.full((LANES,), r, jnp.int32)
            for j in range(VECS_PER_ROW):
                v = rows_v[r, pl.ds(j * LANES, LANES)] + pvecs[j]
                plsc.store_scatter(tr_v, [gvecs[j], cvecs[j], rvec], v)

    pltpu.sync_copy(pos_hbm, pos_v)
    pltpu.sync_copy(idx_hbm.at[:, pl.ds(wid * BBLK, BBLK)], idx_v)
    start_gather(0, 0)

    def t_body(i, carry):
        for b in (0, 1):
            t = 2 * i + b
            wait_gather(b)
            nxt = t + 1

            @pl.when(t >= 1)
            def _():
                wait_out(1 - b)

            @pl.when(nxt < MAX_SEQ)
            def _():
                start_gather(nxt, 1 - b)

            compute(t, b)
            start_out(t, b)
        return carry

    lax.fori_loop(0, MAX_SEQ // 2, t_body, 0)
    wait_out((MAX_SEQ - 1) % 2)


@jax.jit
def _embed(idx_t, table_flat, position_table):
    mesh = plsc.VectorSubcoreMesh(core_axis_name="c", subcore_axis_name="s")
    table_lin = jnp.reshape(table_flat, (VOCAB, EMBED))
    run = pl.kernel(
        _embed_body,
        out_type=jax.ShapeDtypeStruct(
            (MAX_SEQ, CGRP, NUM_WORKERS, 8, BBLK), jnp.float32),
        mesh=mesh,
        scratch_types=[
            pltpu.VMEM((MAX_SEQ, BBLK), jnp.int32),
            pltpu.VMEM((BBLK, EMBED), jnp.float32),
            pltpu.VMEM((BBLK, EMBED), jnp.float32),
            pltpu.VMEM((CGRP, 8, BBLK), jnp.float32),
            pltpu.VMEM((CGRP, 8, BBLK), jnp.float32),
            pltpu.VMEM((MAX_SEQ, EMBED), jnp.float32),
            pltpu.SemaphoreType.DMA,
            pltpu.SemaphoreType.DMA,
            pltpu.SemaphoreType.DMA,
            pltpu.SemaphoreType.DMA,
        ],
        compiler_params=pltpu.CompilerParams(
            use_tc_tiling_on_sc=False, needs_layout_passes=False),
    )
    return run(table_lin, idx_t, position_table)


def kernel(inputs, token_table, position_table):
    idx_t = inputs.T.astype(jnp.int32)
    tbl_flat = _transpose_table(token_table.T)
    out5 = _embed(idx_t, tbl_flat, position_table)
    return out5.transpose(2, 4, 0, 1, 3).reshape(BATCH, MAX_SEQ, EMBED)


# speedup vs baseline: 1.2023x; 1.2023x over previous
import jax
import jax.numpy as jnp
from jax import lax
from jax.experimental import pallas as pl
from jax.experimental.pallas import tpu as pltpu
from jax.experimental.pallas import tpu_sc as plsc

BATCH = 4096
MAX_SEQ = 200
EMBED = 64
VOCAB = 1000000
LANES = 16

_info = plsc.get_sparse_core_info()
NUM_CORES = _info.num_cores
NUM_SUBCORES = _info.num_subcores
NUM_WORKERS = NUM_CORES * NUM_SUBCORES

TOTAL_ROWS = BATCH * MAX_SEQ
ROWS_PER_WORKER = TOTAL_ROWS // NUM_WORKERS
CHUNK = 512
NCHUNKS = ROWS_PER_WORKER // CHUNK


def _gather_body(table_hbm, idx_hbm, out_hbm,
                 idx0, idx1, rows0, rows1,
                 is0, is1, gs0, gs1, os0, os1):
    wid = lax.axis_index("s") * NUM_CORES + lax.axis_index("c")
    base = wid * ROWS_PER_WORKER

    bufs = ((idx0, rows0, is0, gs0, os0),
            (idx1, rows1, is1, gs1, os1))

    def start_idx(g, b):
        idx_v, _, isem, _, _ = bufs[b]
        pltpu.async_copy(idx_hbm.at[pl.ds(base + g * CHUNK, CHUNK)], idx_v, isem)

    def wait_idx(b):
        idx_v, _, isem, _, _ = bufs[b]
        pltpu.make_async_copy(idx_hbm.at[pl.ds(base, CHUNK)], idx_v, isem).wait()

    def start_gather(b):
        idx_v, rows_v, _, gsem, _ = bufs[b]
        pltpu.async_copy(table_hbm.at[idx_v], rows_v, gsem)

    def wait_gather(b):
        idx_v, rows_v, _, gsem, _ = bufs[b]
        pltpu.make_async_copy(table_hbm.at[idx_v], rows_v, gsem).wait()

    def start_scatter(g, b):
        _, rows_v, _, _, osem = bufs[b]
        pltpu.async_copy(rows_v, out_hbm.at[pl.ds(base + g * CHUNK, CHUNK)], osem)

    def wait_scatter(b):
        _, rows_v, _, _, osem = bufs[b]
        pltpu.make_async_copy(rows_v, out_hbm.at[pl.ds(base, CHUNK)], osem).wait()

    start_idx(0, 0)
    start_idx(1, 1)
    wait_idx(0)
    start_gather(0)

    def pair_body(i, carry):
        for b in (0, 1):
            g = 2 * i + b
            wait_gather(b)
            @pl.when(g + 2 < NCHUNKS)
            def _():
                start_idx(g + 2, b)
            @pl.when(g >= 1)
            def _():
                wait_scatter(1 - b)
            @pl.when(g + 1 < NCHUNKS)
            def _():
                wait_idx(1 - b)
                start_gather(1 - b)
            start_scatter(g, b)
        return carry

    lax.fori_loop(0, NCHUNKS // 2, pair_body, 0)
    wait_scatter((NCHUNKS - 1) % 2)


@jax.jit
def _gather(idx_flat, token_table):
    mesh = plsc.VectorSubcoreMesh(core_axis_name="c", subcore_axis_name="s")
    run = pl.kernel(
        _gather_body,
        out_type=jax.ShapeDtypeStruct((TOTAL_ROWS, EMBED), jnp.float32),
        mesh=mesh,
        scratch_types=[
            pltpu.VMEM((CHUNK,), jnp.int32),
            pltpu.VMEM((CHUNK,), jnp.int32),
            pltpu.VMEM((CHUNK, EMBED), jnp.float32),
            pltpu.VMEM((CHUNK, EMBED), jnp.float32),
            pltpu.SemaphoreType.DMA,
            pltpu.SemaphoreType.DMA,
            pltpu.SemaphoreType.DMA,
            pltpu.SemaphoreType.DMA,
            pltpu.SemaphoreType.DMA,
            pltpu.SemaphoreType.DMA,
        ],
        compiler_params=pltpu.CompilerParams(use_tc_tiling_on_sc=False),
    )
    return run(token_table, idx_flat)


def kernel(inputs, token_table, position_table):
    idx_flat = inputs.reshape(-1).astype(jnp.int32)
    gathered = _gather(idx_flat, token_table)
    return gathered.reshape(BATCH, MAX_SEQ, EMBED) + position_table[None, :, :]


# speedup vs baseline: 1.2888x; 1.0720x over previous
import jax
import jax.numpy as jnp
from jax import lax
from jax.experimental import pallas as pl
from jax.experimental.pallas import tpu as pltpu
from jax.experimental.pallas import tpu_sc as plsc

BATCH = 4096
MAX_SEQ = 200
EMBED = 64
VOCAB = 1000000
LANES = 16

_info = plsc.get_sparse_core_info()
NUM_CORES = _info.num_cores
NUM_SUBCORES = _info.num_subcores
NUM_WORKERS = NUM_CORES * NUM_SUBCORES

TOTAL_ROWS = BATCH * MAX_SEQ
ROWS_PER_WORKER = TOTAL_ROWS // NUM_WORKERS
CHUNK = 400
NCHUNKS = ROWS_PER_WORKER // CHUNK
VECS_PER_ROW = EMBED // LANES


def _gather_body(table_hbm, idx_hbm, pos_hbm, out_hbm,
                 idx0, idx1, rows0, rows1, pos_v,
                 is0, is1, gs0, gs1, os0, os1):
    wid = lax.axis_index("s") * NUM_CORES + lax.axis_index("c")
    base = wid * ROWS_PER_WORKER

    bufs = ((idx0, rows0, is0, gs0, os0),
            (idx1, rows1, is1, gs1, os1))

    def start_idx(g, b):
        idx_v, _, isem, _, _ = bufs[b]
        pltpu.async_copy(idx_hbm.at[pl.ds(base + g * CHUNK, CHUNK)], idx_v, isem)

    def wait_idx(b):
        idx_v, _, isem, _, _ = bufs[b]
        pltpu.make_async_copy(idx_hbm.at[pl.ds(base, CHUNK)], idx_v, isem).wait()

    def start_gather(b):
        idx_v, rows_v, _, gsem, _ = bufs[b]
        pltpu.async_copy(table_hbm.at[idx_v], rows_v, gsem)

    def wait_gather(b):
        idx_v, rows_v, _, gsem, _ = bufs[b]
        pltpu.make_async_copy(table_hbm.at[idx_v], rows_v, gsem).wait()

    def start_scatter(g, b):
        _, rows_v, _, _, osem = bufs[b]
        pltpu.async_copy(rows_v, out_hbm.at[pl.ds(base + g * CHUNK, CHUNK)], osem)

    def wait_scatter(b):
        _, rows_v, _, _, osem = bufs[b]
        pltpu.make_async_copy(rows_v, out_hbm.at[pl.ds(base, CHUNK)], osem).wait()

    def add_pos(b):
        _, rows_v, _, _, _ = bufs[b]

        @plsc.parallel_loop(0, CHUNK, 1, unroll=8)
        def _body(r):
            for j in range(VECS_PER_ROW):
                sl = pl.ds(j * LANES, LANES)
                plsc.addupdate(rows_v.at[r, sl], pos_v[r, sl])

    pltpu.sync_copy(pos_hbm, pos_v)
    start_idx(0, 0)
    start_idx(1, 1)
    wait_idx(0)
    start_gather(0)

    def pair_body(i, carry):
        for b in (0, 1):
            g = 2 * i + b
            wait_gather(b)
            @pl.when(g + 2 < NCHUNKS)
            def _():
                start_idx(g + 2, b)
            @pl.when(g >= 1)
            def _():
                wait_scatter(1 - b)
            @pl.when(g + 1 < NCHUNKS)
            def _():
                wait_idx(1 - b)
                start_gather(1 - b)
            add_pos(b)
            start_scatter(g, b)
        return carry

    lax.fori_loop(0, NCHUNKS // 2, pair_body, 0)
    wait_scatter((NCHUNKS - 1) % 2)


@jax.jit
def _gather(idx_flat, token_table, pos_tiled):
    mesh = plsc.VectorSubcoreMesh(core_axis_name="c", subcore_axis_name="s")
    run = pl.kernel(
        _gather_body,
        out_type=jax.ShapeDtypeStruct((TOTAL_ROWS, EMBED), jnp.float32),
        mesh=mesh,
        scratch_types=[
            pltpu.VMEM((CHUNK,), jnp.int32),
            pltpu.VMEM((CHUNK,), jnp.int32),
            pltpu.VMEM((CHUNK, EMBED), jnp.float32),
            pltpu.VMEM((CHUNK, EMBED), jnp.float32),
            pltpu.VMEM((CHUNK, EMBED), jnp.float32),
            pltpu.SemaphoreType.DMA,
            pltpu.SemaphoreType.DMA,
            pltpu.SemaphoreType.DMA,
            pltpu.SemaphoreType.DMA,
            pltpu.SemaphoreType.DMA,
            pltpu.SemaphoreType.DMA,
        ],
        compiler_params=pltpu.CompilerParams(use_tc_tiling_on_sc=False),
    )
    return run(token_table, idx_flat, pos_tiled)


def kernel(inputs, token_table, position_table):
    idx_flat = inputs.reshape(-1).astype(jnp.int32)
    pos_tiled = jnp.tile(position_table, (CHUNK // MAX_SEQ, 1))
    out = _gather(idx_flat, token_table, pos_tiled)
    return out.reshape(BATCH, MAX_SEQ, EMBED)
